# baseline (speedup 1.0000x reference)
import functools

import jax
import jax.numpy as jnp
from jax import lax
from jax.experimental import pallas as pl
from jax.experimental.pallas import tpu as pltpu
from jax.experimental.pallas import tpu_sc as plsc

N = 10000
E_EDGES = 320000
L = 16
V = 30000
D = 128
H = 128
HEADS = 8
G = 500
C = 2
SEM_H = 128

NW = 32
RNG = 16
NRANGES = 640
RPT = NRANGES // NW
GH = 3 * HEADS
MW = GH * H


def _reverse_padded(x, lengths):
    T = x.shape[1]
    t = jnp.arange(T)[None, :]
    idx = jnp.where(t < lengths[:, None], lengths[:, None] - 1 - t, t)
    return jnp.take_along_axis(x, idx[:, :, None], axis=1)


def _lstm_dir(x, lengths, p):
    n = x.shape[0]
    h_dim = p['Whh'].shape[1]
    xs = jnp.transpose(x, (1, 0, 2))
    mask = (jnp.arange(x.shape[1])[:, None] < lengths[None, :]).astype(x.dtype)

    def step(carry, inp):
        h, c = carry
        xt, m = inp
        dt = h.dtype
        gates = (xt @ p['Wih'].T + p['bih'] + h @ p['Whh'].T + p['bhh']).astype(dt)
        i, f, g, o = jnp.split(gates, 4, axis=-1)
        i = jax.nn.sigmoid(i)
        f = jax.nn.sigmoid(f)
        o = jax.nn.sigmoid(o)
        g = jnp.tanh(g)
        cn = f * c + i * g
        hn = o * jnp.tanh(cn)
        m2 = m[:, None]
        h = (m2 * hn + (1.0 - m2) * h).astype(dt)
        c = (m2 * cn + (1.0 - m2) * c).astype(dt)
        return (h, c), h

    init = (jnp.zeros((n, h_dim), x.dtype), jnp.zeros((n, h_dim), x.dtype))
    (hf, _), hs = jax.lax.scan(step, init, (xs, mask))
    return jnp.transpose(hs, (1, 0, 2)), hf


def _bilstm(x, lengths, pf, pb):
    of, hf = _lstm_dir(x, lengths, pf)
    xr = _reverse_padded(x, lengths)
    obr, hb = _lstm_dir(xr, lengths, pb)
    ob = _reverse_padded(obr, lengths)
    return jnp.concatenate([of, ob], axis=-1), hf, hb


def _edge_body(dst_hbm, src_hbm, eb_hbm, asrc_hbm, adst_hbm, xw_hbm, out_hbm,
               ebrow, srcbuf, dstbuf, arows_s, arows_d, msgbuf, slab, den,
               abuf, sem_a, sem_b, sem_m):
    cid = lax.axis_index("c")
    sid = lax.axis_index("s")
    wid = sid * 2 + cid
    lane = lax.iota(jnp.int32, 16)
    zero16 = jnp.zeros((16,), jnp.float32)


    def edge_softmax_terms(j):
        s_lo = arows_s[j, 0:16]
        s_hi = arows_s[j, 16:32]
        d_lo = arows_d[j, 0:16]
        d_hi = arows_d[j, 16:32]
        e_lo = s_lo + d_lo
        e_hi = s_hi + d_hi
        e_lo = jnp.where(e_lo < 0.0, e_lo * 0.2, e_lo)
        e_hi = jnp.where(e_hi < 0.0, e_hi * 0.2, e_hi)
        return jnp.exp(e_lo), jnp.exp(e_hi)

    def range_body(rr, _):
        r = rr * NW + wid
        base = r * RNG
        pltpu.sync_copy(eb_hbm.at[r], ebrow)
        ebv = ebrow[...]
        e_start = ebv[0]
        e_end = ebv[1]
        k0 = (e_start // 16) * 16
        nb = jnp.maximum(e_end - k0 + 15, 0) // 16

        def zrow(i, _):
            def zc(c, _):
                slab[i, pl.ds(c * 16, 16)] = zero16
                return 0
            lax.fori_loop(0, MW // 16, zc, 0)
            den[i, 0:16] = zero16
            den[i, 16:32] = zero16
            return 0
        lax.fori_loop(0, RNG, zrow, 0)

        def stage_batch(b):
            e0 = k0 + b * 16
            pltpu.sync_copy(src_hbm.at[pl.ds(e0, 16)], srcbuf)
            pltpu.sync_copy(dst_hbm.at[pl.ds(e0, 16)], dstbuf)
            pltpu.async_copy(asrc_hbm.at[srcbuf], arows_s, sem_a).wait()
            pltpu.async_copy(adst_hbm.at[dstbuf], arows_d, sem_b).wait()
            return e0

        def dloc_of(j):
            dsp = plsc.load_gather(dstbuf, [jnp.full((16,), j, jnp.int32)])
            return dsp & (RNG - 1)

        def pass_a(b, _):
            e0 = stage_batch(b)

            def pe(j, _):
                valid = (e0 + j >= e_start) & (e0 + j < e_end)
                ex_lo, ex_hi = edge_softmax_terms(j)
                vf = jnp.where(valid, 1.0, 0.0)
                dloc_v = dloc_of(j)
                plsc.addupdate_scatter(den, [dloc_v, lane], ex_lo * vf)
                plsc.addupdate_scatter(den, [dloc_v, lane + 16], ex_hi * vf)
                return 0
            lax.fori_loop(0, 16, pe, 0)
            return 0
        lax.fori_loop(0, nb, pass_a, 0)

        def pass_b(b, _):
            e0 = stage_batch(b)
            pltpu.async_copy(xw_hbm.at[srcbuf], msgbuf, sem_m).wait()

            def pe(j, _):
                valid = (e0 + j >= e_start) & (e0 + j < e_end)
                ex_lo, ex_hi = edge_softmax_terms(j)
                vf = jnp.where(valid, 1.0, 0.0)
                dloc_v = dloc_of(j)
                den_lo = plsc.load_gather(den, [dloc_v, lane])
                den_hi = plsc.load_gather(den, [dloc_v, lane + 16])
                abuf[0:16] = ex_lo / (den_lo + 1e-16) * vf
                abuf[16:32] = ex_hi / (den_hi + 1e-16) * vf
                for g in range(GH):
                    a_g = plsc.load_gather(abuf, [jnp.full((16,), g, jnp.int32)])
                    for c in range(H // 16):
                        off = g * H + c * 16
                        plsc.addupdate_scatter(
                            slab, [dloc_v, lane + off],
                            msgbuf[j, pl.ds(off, 16)] * a_g)
                return 0
            lax.fori_loop(0, 16, pe, 0)
            return 0
        lax.fori_loop(0, nb, pass_b, 0)

        @pl.when(base < N)
        def _flush():
            pltpu.sync_copy(slab, out_hbm.at[pl.ds(base, RNG)])
        return 0

    lax.fori_loop(0, RPT, range_body, 0)


_edge_kernel = functools.partial(
    pl.kernel,
    out_type=jax.ShapeDtypeStruct((N, MW), jnp.float32),
    mesh=plsc.VectorSubcoreMesh(core_axis_name="c", subcore_axis_name="s"),
    compiler_params=pltpu.CompilerParams(needs_layout_passes=False),
    scratch_types=[
        pltpu.VMEM((16,), jnp.int32),
        pltpu.VMEM((16,), jnp.int32),
        pltpu.VMEM((16,), jnp.int32),
        pltpu.VMEM((16, 128), jnp.float32),
        pltpu.VMEM((16, 128), jnp.float32),
        pltpu.VMEM((16, MW), jnp.float32),
        pltpu.VMEM((RNG, MW), jnp.float32),
        pltpu.VMEM((RNG, 32), jnp.float32),
        pltpu.VMEM((32,), jnp.float32),
        pltpu.SemaphoreType.DMA,
        pltpu.SemaphoreType.DMA,
        pltpu.SemaphoreType.DMA,
    ],
)(_edge_body)


def _gat3_sparse(node, edge_index, p1, p2, p3):
    src = edge_index[0]
    dst = edge_index[1]
    order = jnp.argsort(dst)
    dst_s = jnp.concatenate([dst[order], jnp.zeros((16,), dst.dtype)])
    src_s = jnp.concatenate([src[order], jnp.zeros((16,), src.dtype)])
    bounds = jnp.searchsorted(dst_s[:E_EDGES], jnp.arange(NRANGES + 1) * RNG)
    ebmat = jnp.zeros((NRANGES, 16), jnp.int32)
    ebmat = ebmat.at[:, 0].set(bounds[:NRANGES].astype(jnp.int32))
    ebmat = ebmat.at[:, 1].set(bounds[1:].astype(jnp.int32))

    xws = []
    asrcs = []
    adsts = []
    for p in (p1, p2, p3):
        xw = node @ p['W']
        xwh = xw.reshape(N, HEADS, H)
        asrcs.append((xwh * p['att_src'][None]).sum(-1))
        adsts.append((xwh * p['att_dst'][None]).sum(-1))
        xws.append(xw)
    xw_cat = jnp.concatenate(xws, axis=1)
    pad = jnp.zeros((N, 128 - GH), jnp.float32)
    asrc_cat = jnp.concatenate(asrcs + [pad], axis=1)
    adst_cat = jnp.concatenate(adsts + [pad], axis=1)

    out_cat = _edge_kernel(dst_s.astype(jnp.int32), src_s.astype(jnp.int32),
                           ebmat, asrc_cat, adst_cat, xw_cat)
    outs = []
    for i, p in enumerate((p1, p2, p3)):
        outs.append(jax.nn.relu(out_cat[:, i * HEADS * H:(i + 1) * HEADS * H]
                                + p['bias']))
    return outs


def _head_body(pooled_ref, w_ref, b_ref, labels_ref, logits_ref, loss_ref):
    logits = jnp.dot(pooled_ref[...], w_ref[...],
                     preferred_element_type=jnp.float32) + b_ref[...]
    m = jnp.max(logits, axis=-1, keepdims=True)
    lse = m + jnp.log(jnp.sum(jnp.exp(logits - m), axis=-1, keepdims=True))
    logp = logits - lse
    cols = lax.broadcasted_iota(jnp.int32, logp.shape, 1)
    pick = jnp.sum(jnp.where(cols == labels_ref[...], logp, 0.0), axis=-1)
    loss_ref[...] = jnp.reshape(-jnp.mean(pick), (1, 1))
    logits_ref[...] = logits


def _head(pooled, w, b, labels):
    logits, loss = pl.pallas_call(
        _head_body,
        out_shape=(
            jax.ShapeDtypeStruct((G, C), jnp.float32),
            jax.ShapeDtypeStruct((1, 1), jnp.float32),
        ),
    )(pooled, w, b.reshape(1, C), labels.reshape(G, 1).astype(jnp.int32))
    return loss.reshape(()), logits


def kernel(params, x, edge_index, batch, labels):
    input_ids = x[:, 0, :]
    attn = x[:, 1, :]
    lengths = attn.sum(axis=-1)
    emb = jnp.take(params['emb'], input_ids, axis=0)
    out0, hf0, hb0 = _bilstm(emb, lengths, params['lstm']['l0f'], params['lstm']['l0b'])
    out1, hf1, hb1 = _bilstm(out0, lengths, params['lstm']['l1f'], params['lstm']['l1b'])
    node = (hf0 + hb0 + hf1 + hb1) / 4.0
    sems = _gat3_sparse(node, edge_index, params['gat1'], params['gat2'], params['gat3'])
    z = jnp.stack(sems, axis=1)
    w = jnp.tanh(z @ params['sem']['W1'] + params['sem']['b1']) @ params['sem']['W2']
    beta = jax.nn.softmax(w.mean(axis=0), axis=0)
    sem_emb = (beta[None] * z).sum(axis=1)
    ng = labels.shape[0]
    sums = jax.ops.segment_sum(sem_emb, batch, num_segments=ng)
    cnt = jax.ops.segment_sum(jnp.ones((sem_emb.shape[0],), sem_emb.dtype), batch, num_segments=ng)
    pooled = sums / jnp.clip(cnt, 1.0)[:, None]
    loss, logits = _head(pooled, params['cls']['W'], params['cls']['b'], labels)
    return loss, logits


# speedup vs baseline: 1.6123x; 1.6123x over previous
import functools

import jax
import jax.numpy as jnp
from jax import lax
from jax.experimental import pallas as pl
from jax.experimental.pallas import tpu as pltpu
from jax.experimental.pallas import tpu_sc as plsc

N = 10000
E_EDGES = 320000
L = 16
V = 30000
D = 128
H = 128
HEADS = 8
G = 500
C = 2
SEM_H = 128

NW = 32
RNG = 16
NRANGES = 640
RPT = NRANGES // NW
GH = 3 * HEADS
MW = GH * H


BLK = 400


def _lstm_step(x_t, h, c, wih_t, whh_t, b, m):
    g = jnp.dot(x_t, wih_t, preferred_element_type=jnp.float32) \
        + jnp.dot(h, whh_t, preferred_element_type=jnp.float32) + b
    i = jax.nn.sigmoid(g[:, 0:H])
    f = jax.nn.sigmoid(g[:, H:2 * H])
    gg = jnp.tanh(g[:, 2 * H:3 * H])
    o = jax.nn.sigmoid(g[:, 3 * H:4 * H])
    cn = f * c + i * gg
    hn = o * jnp.tanh(cn)
    h = m * hn + (1.0 - m) * h
    c = m * cn + (1.0 - m) * c
    return h, c


def _bilstm_body(emb_ref, len_ref, w0f_ref, w0b_ref, wh0f_ref, wh0b_ref,
                 b0f_ref, b0b_ref, w1f_ref, w1b_ref, wh1f_ref, wh1b_ref,
                 b1f_ref, b1b_ref, node_ref, of_scr, ob_scr):
    lens = len_ref[...]
    zero = jnp.zeros((emb_ref.shape[0], H), jnp.float32)

    def run_dir(get_x, w_t, wh_t, b, reverse, out_scr):
        h, c = zero, zero
        ts = range(L - 1, -1, -1) if reverse else range(L)
        for t in ts:
            m = (lens > t).astype(jnp.float32)
            h, c = _lstm_step(get_x(t), h, c, w_t, wh_t, b, m)
            if out_scr is not None:
                out_scr[:, t, :] = h
        return h

    x0 = lambda t: emb_ref[:, t, :]
    hf0 = run_dir(x0, w0f_ref[...], wh0f_ref[...], b0f_ref[...], False, of_scr)
    hb0 = run_dir(x0, w0b_ref[...], wh0b_ref[...], b0b_ref[...], True, ob_scr)
    x1 = lambda t: jnp.concatenate([of_scr[:, t, :], ob_scr[:, t, :]], axis=1)
    hf1 = run_dir(x1, w1f_ref[...], wh1f_ref[...], b1f_ref[...], False, None)
    hb1 = run_dir(x1, w1b_ref[...], wh1b_ref[...], b1b_ref[...], True, None)
    node_ref[...] = (hf0 + hb0 + hf1 + hb1) * 0.25


def _bilstm_node(emb, lengths, lstm):
    nb = N // BLK
    full = lambda i: (0, 0)
    return pl.pallas_call(
        _bilstm_body,
        grid=(nb,),
        in_specs=[
            pl.BlockSpec((BLK, L, D), lambda i: (i, 0, 0)),
            pl.BlockSpec((BLK, 1), lambda i: (i, 0)),
            pl.BlockSpec((D, 4 * H), full),
            pl.BlockSpec((D, 4 * H), full),
            pl.BlockSpec((H, 4 * H), full),
            pl.BlockSpec((H, 4 * H), full),
            pl.BlockSpec((1, 4 * H), full),
            pl.BlockSpec((1, 4 * H), full),
            pl.BlockSpec((2 * H, 4 * H), full),
            pl.BlockSpec((2 * H, 4 * H), full),
            pl.BlockSpec((H, 4 * H), full),
            pl.BlockSpec((H, 4 * H), full),
            pl.BlockSpec((1, 4 * H), full),
            pl.BlockSpec((1, 4 * H), full),
        ],
        out_specs=pl.BlockSpec((BLK, H), lambda i: (i, 0)),
        out_shape=jax.ShapeDtypeStruct((N, H), jnp.float32),
        scratch_shapes=[pltpu.VMEM((BLK, L, H), jnp.float32),
                        pltpu.VMEM((BLK, L, H), jnp.float32)],
    )(
        emb, lengths.reshape(N, 1).astype(jnp.int32),
        lstm['l0f']['Wih'].T, lstm['l0b']['Wih'].T,
        lstm['l0f']['Whh'].T, lstm['l0b']['Whh'].T,
        (lstm['l0f']['bih'] + lstm['l0f']['bhh']).reshape(1, 4 * H),
        (lstm['l0b']['bih'] + lstm['l0b']['bhh']).reshape(1, 4 * H),
        lstm['l1f']['Wih'].T, lstm['l1b']['Wih'].T,
        lstm['l1f']['Whh'].T, lstm['l1b']['Whh'].T,
        (lstm['l1f']['bih'] + lstm['l1f']['bhh']).reshape(1, 4 * H),
        (lstm['l1b']['bih'] + lstm['l1b']['bhh']).reshape(1, 4 * H),
    )


def _edge_body(dst_hbm, src_hbm, eb_hbm, asrc_hbm, adst_hbm, xw_hbm, out_hbm,
               ebrow, srcbuf, dstbuf, arows_s, arows_d, msgbuf, slab, den,
               abuf, sem_a, sem_b, sem_m):
    cid = lax.axis_index("c")
    sid = lax.axis_index("s")
    wid = sid * 2 + cid
    lane = lax.iota(jnp.int32, 16)
    zero16 = jnp.zeros((16,), jnp.float32)


    def edge_softmax_terms(j):
        s_lo = arows_s[j, 0:16]
        s_hi = arows_s[j, 16:32]
        d_lo = arows_d[j, 0:16]
        d_hi = arows_d[j, 16:32]
        e_lo = s_lo + d_lo
        e_hi = s_hi + d_hi
        e_lo = jnp.where(e_lo < 0.0, e_lo * 0.2, e_lo)
        e_hi = jnp.where(e_hi < 0.0, e_hi * 0.2, e_hi)
        return jnp.exp(e_lo), jnp.exp(e_hi)

    def range_body(rr, _):
        r = rr * NW + wid
        base = r * RNG
        pltpu.sync_copy(eb_hbm.at[r], ebrow)
        ebv = ebrow[...]
        e_start = ebv[0]
        e_end = ebv[1]
        k0 = (e_start // 16) * 16
        nb = jnp.maximum(e_end - k0 + 15, 0) // 16

        def zrow(i, _):
            def zc(c, _):
                slab[i, pl.ds(c * 16, 16)] = zero16
                return 0
            lax.fori_loop(0, MW // 16, zc, 0)
            den[i, 0:16] = zero16
            den[i, 16:32] = zero16
            return 0
        lax.fori_loop(0, RNG, zrow, 0)

        def stage_batch(b):
            e0 = k0 + b * 16
            pltpu.sync_copy(src_hbm.at[pl.ds(e0, 16)], srcbuf)
            pltpu.sync_copy(dst_hbm.at[pl.ds(e0, 16)], dstbuf)
            pltpu.async_copy(asrc_hbm.at[srcbuf], arows_s, sem_a).wait()
            pltpu.async_copy(adst_hbm.at[dstbuf], arows_d, sem_b).wait()
            return e0

        def dloc_of(j):
            dsp = plsc.load_gather(dstbuf, [jnp.full((16,), j, jnp.int32)])
            return dsp & (RNG - 1)

        def pass_a(b, _):
            e0 = stage_batch(b)

            def pe(j, _):
                valid = (e0 + j >= e_start) & (e0 + j < e_end)
                ex_lo, ex_hi = edge_softmax_terms(j)
                vf = jnp.where(valid, 1.0, 0.0)
                dloc_v = dloc_of(j)
                plsc.addupdate_scatter(den, [dloc_v, lane], ex_lo * vf)
                plsc.addupdate_scatter(den, [dloc_v, lane + 16], ex_hi * vf)
                return 0
            lax.fori_loop(0, 16, pe, 0)
            return 0
        lax.fori_loop(0, nb, pass_a, 0)

        def pass_b(b, _):
            e0 = stage_batch(b)
            pltpu.async_copy(xw_hbm.at[srcbuf], msgbuf, sem_m).wait()

            def pe(j, _):
                valid = (e0 + j >= e_start) & (e0 + j < e_end)
                ex_lo, ex_hi = edge_softmax_terms(j)
                vf = jnp.where(valid, 1.0, 0.0)
                dloc_v = dloc_of(j)
                den_lo = plsc.load_gather(den, [dloc_v, lane])
                den_hi = plsc.load_gather(den, [dloc_v, lane + 16])
                abuf[0:16] = ex_lo / (den_lo + 1e-16) * vf
                abuf[16:32] = ex_hi / (den_hi + 1e-16) * vf
                for g in range(GH):
                    a_g = plsc.load_gather(abuf, [jnp.full((16,), g, jnp.int32)])
                    for c in range(H // 16):
                        off = g * H + c * 16
                        plsc.addupdate_scatter(
                            slab, [dloc_v, lane + off],
                            msgbuf[j, pl.ds(off, 16)] * a_g)
                return 0
            lax.fori_loop(0, 16, pe, 0)
            return 0
        lax.fori_loop(0, nb, pass_b, 0)

        @pl.when(base < N)
        def _flush():
            pltpu.sync_copy(slab, out_hbm.at[pl.ds(base, RNG)])
        return 0

    lax.fori_loop(0, RPT, range_body, 0)


_edge_kernel = functools.partial(
    pl.kernel,
    out_type=jax.ShapeDtypeStruct((N, MW), jnp.float32),
    mesh=plsc.VectorSubcoreMesh(core_axis_name="c", subcore_axis_name="s"),
    compiler_params=pltpu.CompilerParams(needs_layout_passes=False),
    scratch_types=[
        pltpu.VMEM((16,), jnp.int32),
        pltpu.VMEM((16,), jnp.int32),
        pltpu.VMEM((16,), jnp.int32),
        pltpu.VMEM((16, 128), jnp.float32),
        pltpu.VMEM((16, 128), jnp.float32),
        pltpu.VMEM((16, MW), jnp.float32),
        pltpu.VMEM((RNG, MW), jnp.float32),
        pltpu.VMEM((RNG, 32), jnp.float32),
        pltpu.VMEM((32,), jnp.float32),
        pltpu.SemaphoreType.DMA,
        pltpu.SemaphoreType.DMA,
        pltpu.SemaphoreType.DMA,
    ],
)(_edge_body)


def _gat3_sparse(node, edge_index, p1, p2, p3):
    src = edge_index[0]
    dst = edge_index[1]
    order = jnp.argsort(dst)
    dst_s = jnp.concatenate([dst[order], jnp.zeros((16,), dst.dtype)])
    src_s = jnp.concatenate([src[order], jnp.zeros((16,), src.dtype)])
    bounds = jnp.searchsorted(dst_s[:E_EDGES], jnp.arange(NRANGES + 1) * RNG)
    ebmat = jnp.zeros((NRANGES, 16), jnp.int32)
    ebmat = ebmat.at[:, 0].set(bounds[:NRANGES].astype(jnp.int32))
    ebmat = ebmat.at[:, 1].set(bounds[1:].astype(jnp.int32))

    xws = []
    asrcs = []
    adsts = []
    for p in (p1, p2, p3):
        xw = node @ p['W']
        xwh = xw.reshape(N, HEADS, H)
        asrcs.append((xwh * p['att_src'][None]).sum(-1))
        adsts.append((xwh * p['att_dst'][None]).sum(-1))
        xws.append(xw)
    xw_cat = jnp.concatenate(xws, axis=1)
    pad = jnp.zeros((N, 128 - GH), jnp.float32)
    asrc_cat = jnp.concatenate(asrcs + [pad], axis=1)
    adst_cat = jnp.concatenate(adsts + [pad], axis=1)

    out_cat = _edge_kernel(dst_s.astype(jnp.int32), src_s.astype(jnp.int32),
                           ebmat, asrc_cat, adst_cat, xw_cat)
    outs = []
    for i, p in enumerate((p1, p2, p3)):
        outs.append(jax.nn.relu(out_cat[:, i * HEADS * H:(i + 1) * HEADS * H]
                                + p['bias']))
    return outs


def _head_body(pooled_ref, w_ref, b_ref, labels_ref, logits_ref, loss_ref):
    logits = jnp.dot(pooled_ref[...], w_ref[...],
                     preferred_element_type=jnp.float32) + b_ref[...]
    m = jnp.max(logits, axis=-1, keepdims=True)
    lse = m + jnp.log(jnp.sum(jnp.exp(logits - m), axis=-1, keepdims=True))
    logp = logits - lse
    cols = lax.broadcasted_iota(jnp.int32, logp.shape, 1)
    pick = jnp.sum(jnp.where(cols == labels_ref[...], logp, 0.0), axis=-1)
    loss_ref[...] = jnp.reshape(-jnp.mean(pick), (1, 1))
    logits_ref[...] = logits


def _head(pooled, w, b, labels):
    logits, loss = pl.pallas_call(
        _head_body,
        out_shape=(
            jax.ShapeDtypeStruct((G, C), jnp.float32),
            jax.ShapeDtypeStruct((1, 1), jnp.float32),
        ),
    )(pooled, w, b.reshape(1, C), labels.reshape(G, 1).astype(jnp.int32))
    return loss.reshape(()), logits


def kernel(params, x, edge_index, batch, labels):
    input_ids = x[:, 0, :]
    attn = x[:, 1, :]
    lengths = attn.sum(axis=-1)
    emb = jnp.take(params['emb'], input_ids, axis=0)
    node = _bilstm_node(emb, lengths, params['lstm'])
    sems = _gat3_sparse(node, edge_index, params['gat1'], params['gat2'], params['gat3'])
    z = jnp.stack(sems, axis=1)
    w = jnp.tanh(z @ params['sem']['W1'] + params['sem']['b1']) @ params['sem']['W2']
    beta = jax.nn.softmax(w.mean(axis=0), axis=0)
    sem_emb = (beta[None] * z).sum(axis=1)
    ng = labels.shape[0]
    sums = jax.ops.segment_sum(sem_emb, batch, num_segments=ng)
    cnt = jax.ops.segment_sum(jnp.ones((sem_emb.shape[0],), sem_emb.dtype), batch, num_segments=ng)
    pooled = sums / jnp.clip(cnt, 1.0)[:, None]
    loss, logits = _head(pooled, params['cls']['W'], params['cls']['b'], labels)
    return loss, logits


# speedup vs baseline: 1.8221x; 1.1301x over previous
import functools

import jax
import jax.numpy as jnp
from jax import lax
from jax.experimental import pallas as pl
from jax.experimental.pallas import tpu as pltpu
from jax.experimental.pallas import tpu_sc as plsc

N = 10000
E_EDGES = 320000
L = 16
V = 30000
D = 128
H = 128
HEADS = 8
G = 500
C = 2
SEM_H = 128

NW = 32
RNG = 16
NRANGES = 640
RPT = NRANGES // NW
GH = 3 * HEADS
MW = GH * H


BLK = 400


def _lstm_step(x_t, h, c, wih_t, whh_t, b, m):
    g = jnp.dot(x_t, wih_t, preferred_element_type=jnp.float32) \
        + jnp.dot(h, whh_t, preferred_element_type=jnp.float32) + b
    i = jax.nn.sigmoid(g[:, 0:H])
    f = jax.nn.sigmoid(g[:, H:2 * H])
    gg = jnp.tanh(g[:, 2 * H:3 * H])
    o = jax.nn.sigmoid(g[:, 3 * H:4 * H])
    cn = f * c + i * gg
    hn = o * jnp.tanh(cn)
    h = m * hn + (1.0 - m) * h
    c = m * cn + (1.0 - m) * c
    return h, c


def _bilstm_body(emb_ref, len_ref, w0f_ref, w0b_ref, wh0f_ref, wh0b_ref,
                 b0f_ref, b0b_ref, w1f_ref, w1b_ref, wh1f_ref, wh1b_ref,
                 b1f_ref, b1b_ref, node_ref, of_scr, ob_scr):
    lens = len_ref[...]
    zero = jnp.zeros((emb_ref.shape[0], H), jnp.float32)

    def run_dir(get_x, w_t, wh_t, b, reverse, out_scr):
        h, c = zero, zero
        ts = range(L - 1, -1, -1) if reverse else range(L)
        for t in ts:
            m = (lens > t).astype(jnp.float32)
            h, c = _lstm_step(get_x(t), h, c, w_t, wh_t, b, m)
            if out_scr is not None:
                out_scr[:, t, :] = h
        return h

    x0 = lambda t: emb_ref[:, t, :]
    hf0 = run_dir(x0, w0f_ref[...], wh0f_ref[...], b0f_ref[...], False, of_scr)
    hb0 = run_dir(x0, w0b_ref[...], wh0b_ref[...], b0b_ref[...], True, ob_scr)
    x1 = lambda t: jnp.concatenate([of_scr[:, t, :], ob_scr[:, t, :]], axis=1)
    hf1 = run_dir(x1, w1f_ref[...], wh1f_ref[...], b1f_ref[...], False, None)
    hb1 = run_dir(x1, w1b_ref[...], wh1b_ref[...], b1b_ref[...], True, None)
    node_ref[...] = (hf0 + hb0 + hf1 + hb1) * 0.25


def _bilstm_node(emb, lengths, lstm):
    nb = N // BLK
    full = lambda i: (0, 0)
    return pl.pallas_call(
        _bilstm_body,
        grid=(nb,),
        in_specs=[
            pl.BlockSpec((BLK, L, D), lambda i: (i, 0, 0)),
            pl.BlockSpec((BLK, 1), lambda i: (i, 0)),
            pl.BlockSpec((D, 4 * H), full),
            pl.BlockSpec((D, 4 * H), full),
            pl.BlockSpec((H, 4 * H), full),
            pl.BlockSpec((H, 4 * H), full),
            pl.BlockSpec((1, 4 * H), full),
            pl.BlockSpec((1, 4 * H), full),
            pl.BlockSpec((2 * H, 4 * H), full),
            pl.BlockSpec((2 * H, 4 * H), full),
            pl.BlockSpec((H, 4 * H), full),
            pl.BlockSpec((H, 4 * H), full),
            pl.BlockSpec((1, 4 * H), full),
            pl.BlockSpec((1, 4 * H), full),
        ],
        out_specs=pl.BlockSpec((BLK, H), lambda i: (i, 0)),
        out_shape=jax.ShapeDtypeStruct((N, H), jnp.float32),
        scratch_shapes=[pltpu.VMEM((BLK, L, H), jnp.float32),
                        pltpu.VMEM((BLK, L, H), jnp.float32)],
    )(
        emb, lengths.reshape(N, 1).astype(jnp.int32),
        lstm['l0f']['Wih'].T, lstm['l0b']['Wih'].T,
        lstm['l0f']['Whh'].T, lstm['l0b']['Whh'].T,
        (lstm['l0f']['bih'] + lstm['l0f']['bhh']).reshape(1, 4 * H),
        (lstm['l0b']['bih'] + lstm['l0b']['bhh']).reshape(1, 4 * H),
        lstm['l1f']['Wih'].T, lstm['l1b']['Wih'].T,
        lstm['l1f']['Whh'].T, lstm['l1b']['Whh'].T,
        (lstm['l1f']['bih'] + lstm['l1f']['bhh']).reshape(1, 4 * H),
        (lstm['l1b']['bih'] + lstm['l1b']['bhh']).reshape(1, 4 * H),
    )


def _edge_body(dst_hbm, src_hbm, eb_hbm, asrc_hbm, adst_hbm, xw_hbm, out_hbm,
               ebrow, srcbuf, dstbuf, arows_s, arows_d, msgbuf, slab, den,
               abuf, sem_a, sem_b, sem_m):
    cid = lax.axis_index("c")
    sid = lax.axis_index("s")
    wid = sid * 2 + cid
    lane = lax.iota(jnp.int32, 16)
    zero16 = jnp.zeros((16,), jnp.float32)


    def edge_softmax_terms(j):
        s_lo = arows_s[j, 0:16]
        s_hi = arows_s[j, 16:32]
        d_lo = arows_d[j, 0:16]
        d_hi = arows_d[j, 16:32]
        e_lo = s_lo + d_lo
        e_hi = s_hi + d_hi
        e_lo = jnp.where(e_lo < 0.0, e_lo * 0.2, e_lo)
        e_hi = jnp.where(e_hi < 0.0, e_hi * 0.2, e_hi)
        return jnp.exp(e_lo), jnp.exp(e_hi)

    def range_body(rr, _):
        r = rr * NW + wid
        base = r * RNG
        pltpu.sync_copy(eb_hbm.at[r], ebrow)
        ebv = ebrow[...]
        e_start = ebv[0]
        e_end = ebv[1]
        k0 = (e_start // 16) * 16
        nb = jnp.maximum(e_end - k0 + 15, 0) // 16

        def zrow(i, _):
            def zc(c, _):
                slab[i, pl.ds(c * 16, 16)] = zero16
                return 0
            lax.fori_loop(0, MW // 16, zc, 0)
            den[i, 0:16] = zero16
            den[i, 16:32] = zero16
            return 0
        lax.fori_loop(0, RNG, zrow, 0)

        def stage_batch(b, want_msg):
            e0 = k0 + b * 16
            c1 = pltpu.async_copy(src_hbm.at[pl.ds(e0, 16)], srcbuf, sem_a)
            c2 = pltpu.async_copy(dst_hbm.at[pl.ds(e0, 16)], dstbuf, sem_b)
            c1.wait()
            c2.wait()
            g1 = pltpu.async_copy(asrc_hbm.at[srcbuf], arows_s, sem_a)
            g2 = pltpu.async_copy(adst_hbm.at[dstbuf], arows_d, sem_b)
            g3 = (pltpu.async_copy(xw_hbm.at[srcbuf], msgbuf, sem_m)
                  if want_msg else None)
            g1.wait()
            g2.wait()
            if g3 is not None:
                g3.wait()
            return e0

        def dloc_of(j):
            dsp = plsc.load_gather(dstbuf, [jnp.full((16,), j, jnp.int32)])
            return dsp & (RNG - 1)

        def pass_a(b, _):
            e0 = stage_batch(b, False)

            def pe(j, _):
                valid = (e0 + j >= e_start) & (e0 + j < e_end)
                ex_lo, ex_hi = edge_softmax_terms(j)
                vf = jnp.where(valid, 1.0, 0.0)
                dloc_v = dloc_of(j)
                plsc.addupdate_scatter(den, [dloc_v, lane], ex_lo * vf)
                plsc.addupdate_scatter(den, [dloc_v, lane + 16], ex_hi * vf)
                return 0
            lax.fori_loop(0, 16, pe, 0)
            return 0
        lax.fori_loop(0, nb, pass_a, 0)

        def pass_b(b, _):
            e0 = stage_batch(b, True)

            def pe(j, _):
                valid = (e0 + j >= e_start) & (e0 + j < e_end)
                ex_lo, ex_hi = edge_softmax_terms(j)
                vf = jnp.where(valid, 1.0, 0.0)
                dloc_v = dloc_of(j)
                den_lo = plsc.load_gather(den, [dloc_v, lane])
                den_hi = plsc.load_gather(den, [dloc_v, lane + 16])
                abuf[0:16] = ex_lo / (den_lo + 1e-16) * vf
                abuf[16:32] = ex_hi / (den_hi + 1e-16) * vf
                for g in range(GH):
                    a_g = plsc.load_gather(abuf, [jnp.full((16,), g, jnp.int32)])
                    for c in range(H // 16):
                        off = g * H + c * 16
                        plsc.addupdate_scatter(
                            slab, [dloc_v, lane + off],
                            msgbuf[j, pl.ds(off, 16)] * a_g)
                return 0
            lax.fori_loop(0, 16, pe, 0)
            return 0
        lax.fori_loop(0, nb, pass_b, 0)

        @pl.when(base < N)
        def _flush():
            pltpu.sync_copy(slab, out_hbm.at[pl.ds(base, RNG)])
        return 0

    lax.fori_loop(0, RPT, range_body, 0)


_edge_kernel = functools.partial(
    pl.kernel,
    out_type=jax.ShapeDtypeStruct((N, MW), jnp.float32),
    mesh=plsc.VectorSubcoreMesh(core_axis_name="c", subcore_axis_name="s"),
    compiler_params=pltpu.CompilerParams(needs_layout_passes=False),
    scratch_types=[
        pltpu.VMEM((16,), jnp.int32),
        pltpu.VMEM((16,), jnp.int32),
        pltpu.VMEM((16,), jnp.int32),
        pltpu.VMEM((16, 128), jnp.float32),
        pltpu.VMEM((16, 128), jnp.float32),
        pltpu.VMEM((16, MW), jnp.float32),
        pltpu.VMEM((RNG, MW), jnp.float32),
        pltpu.VMEM((RNG, 32), jnp.float32),
        pltpu.VMEM((32,), jnp.float32),
        pltpu.SemaphoreType.DMA,
        pltpu.SemaphoreType.DMA,
        pltpu.SemaphoreType.DMA,
    ],
)(_edge_body)


def _gat3_sparse(node, edge_index, p1, p2, p3):
    src = edge_index[0]
    dst = edge_index[1]
    order = jnp.argsort(dst)
    dst_s = jnp.concatenate([dst[order], jnp.zeros((16,), dst.dtype)])
    src_s = jnp.concatenate([src[order], jnp.zeros((16,), src.dtype)])
    bounds = jnp.searchsorted(dst_s[:E_EDGES], jnp.arange(NRANGES + 1) * RNG)
    ebmat = jnp.zeros((NRANGES, 16), jnp.int32)
    ebmat = ebmat.at[:, 0].set(bounds[:NRANGES].astype(jnp.int32))
    ebmat = ebmat.at[:, 1].set(bounds[1:].astype(jnp.int32))

    xws = []
    asrcs = []
    adsts = []
    for p in (p1, p2, p3):
        xw = node @ p['W']
        xwh = xw.reshape(N, HEADS, H)
        asrcs.append((xwh * p['att_src'][None]).sum(-1))
        adsts.append((xwh * p['att_dst'][None]).sum(-1))
        xws.append(xw)
    xw_cat = jnp.concatenate(xws, axis=1)
    pad = jnp.zeros((N, 128 - GH), jnp.float32)
    asrc_cat = jnp.concatenate(asrcs + [pad], axis=1)
    adst_cat = jnp.concatenate(adsts + [pad], axis=1)

    out_cat = _edge_kernel(dst_s.astype(jnp.int32), src_s.astype(jnp.int32),
                           ebmat, asrc_cat, adst_cat, xw_cat)
    outs = []
    for i, p in enumerate((p1, p2, p3)):
        outs.append(jax.nn.relu(out_cat[:, i * HEADS * H:(i + 1) * HEADS * H]
                                + p['bias']))
    return outs


def _head_body(pooled_ref, w_ref, b_ref, labels_ref, logits_ref, loss_ref):
    logits = jnp.dot(pooled_ref[...], w_ref[...],
                     preferred_element_type=jnp.float32) + b_ref[...]
    m = jnp.max(logits, axis=-1, keepdims=True)
    lse = m + jnp.log(jnp.sum(jnp.exp(logits - m), axis=-1, keepdims=True))
    logp = logits - lse
    cols = lax.broadcasted_iota(jnp.int32, logp.shape, 1)
    pick = jnp.sum(jnp.where(cols == labels_ref[...], logp, 0.0), axis=-1)
    loss_ref[...] = jnp.reshape(-jnp.mean(pick), (1, 1))
    logits_ref[...] = logits


def _head(pooled, w, b, labels):
    logits, loss = pl.pallas_call(
        _head_body,
        out_shape=(
            jax.ShapeDtypeStruct((G, C), jnp.float32),
            jax.ShapeDtypeStruct((1, 1), jnp.float32),
        ),
    )(pooled, w, b.reshape(1, C), labels.reshape(G, 1).astype(jnp.int32))
    return loss.reshape(()), logits


def kernel(params, x, edge_index, batch, labels):
    input_ids = x[:, 0, :]
    attn = x[:, 1, :]
    lengths = attn.sum(axis=-1)
    emb = jnp.take(params['emb'], input_ids, axis=0)
    node = _bilstm_node(emb, lengths, params['lstm'])
    sems = _gat3_sparse(node, edge_index, params['gat1'], params['gat2'], params['gat3'])
    z = jnp.stack(sems, axis=1)
    w = jnp.tanh(z @ params['sem']['W1'] + params['sem']['b1']) @ params['sem']['W2']
    beta = jax.nn.softmax(w.mean(axis=0), axis=0)
    sem_emb = (beta[None] * z).sum(axis=1)
    ng = labels.shape[0]
    sums = jax.ops.segment_sum(sem_emb, batch, num_segments=ng)
    cnt = jax.ops.segment_sum(jnp.ones((sem_emb.shape[0],), sem_emb.dtype), batch, num_segments=ng)
    pooled = sums / jnp.clip(cnt, 1.0)[:, None]
    loss, logits = _head(pooled, params['cls']['W'], params['cls']['b'], labels)
    return loss, logits


# speedup vs baseline: 2.0286x; 1.1133x over previous
import functools

import jax
import jax.numpy as jnp
from jax import lax
from jax.experimental import pallas as pl
from jax.experimental.pallas import tpu as pltpu
from jax.experimental.pallas import tpu_sc as plsc

N = 10000
E_EDGES = 320000
L = 16
V = 30000
D = 128
H = 128
HEADS = 8
G = 500
C = 2
SEM_H = 128

NW = 32
RNG = 16
NRANGES = 640
RPT = NRANGES // NW
GH = 3 * HEADS
MW = GH * H


BLK = 400


def _lstm_step(x_t, h, c, wih_t, whh_t, b, m):
    g = jnp.dot(x_t, wih_t, preferred_element_type=jnp.float32) \
        + jnp.dot(h, whh_t, preferred_element_type=jnp.float32) + b
    i = jax.nn.sigmoid(g[:, 0:H])
    f = jax.nn.sigmoid(g[:, H:2 * H])
    gg = jnp.tanh(g[:, 2 * H:3 * H])
    o = jax.nn.sigmoid(g[:, 3 * H:4 * H])
    cn = f * c + i * gg
    hn = o * jnp.tanh(cn)
    h = m * hn + (1.0 - m) * h
    c = m * cn + (1.0 - m) * c
    return h, c


def _bilstm_body(emb_ref, len_ref, w0f_ref, w0b_ref, wh0f_ref, wh0b_ref,
                 b0f_ref, b0b_ref, w1f_ref, w1b_ref, wh1f_ref, wh1b_ref,
                 b1f_ref, b1b_ref, node_ref, of_scr, ob_scr):
    lens = len_ref[...]
    zero = jnp.zeros((emb_ref.shape[0], H), jnp.float32)

    def run_dir(get_x, w_t, wh_t, b, reverse, out_scr):
        h, c = zero, zero
        ts = range(L - 1, -1, -1) if reverse else range(L)
        for t in ts:
            m = (lens > t).astype(jnp.float32)
            h, c = _lstm_step(get_x(t), h, c, w_t, wh_t, b, m)
            if out_scr is not None:
                out_scr[:, t, :] = h
        return h

    x0 = lambda t: emb_ref[:, t, :]
    hf0 = run_dir(x0, w0f_ref[...], wh0f_ref[...], b0f_ref[...], False, of_scr)
    hb0 = run_dir(x0, w0b_ref[...], wh0b_ref[...], b0b_ref[...], True, ob_scr)
    x1 = lambda t: jnp.concatenate([of_scr[:, t, :], ob_scr[:, t, :]], axis=1)
    hf1 = run_dir(x1, w1f_ref[...], wh1f_ref[...], b1f_ref[...], False, None)
    hb1 = run_dir(x1, w1b_ref[...], wh1b_ref[...], b1b_ref[...], True, None)
    node_ref[...] = (hf0 + hb0 + hf1 + hb1) * 0.25


def _bilstm_node(emb, lengths, lstm):
    nb = N // BLK
    full = lambda i: (0, 0)
    return pl.pallas_call(
        _bilstm_body,
        grid=(nb,),
        in_specs=[
            pl.BlockSpec((BLK, L, D), lambda i: (i, 0, 0)),
            pl.BlockSpec((BLK, 1), lambda i: (i, 0)),
            pl.BlockSpec((D, 4 * H), full),
            pl.BlockSpec((D, 4 * H), full),
            pl.BlockSpec((H, 4 * H), full),
            pl.BlockSpec((H, 4 * H), full),
            pl.BlockSpec((1, 4 * H), full),
            pl.BlockSpec((1, 4 * H), full),
            pl.BlockSpec((2 * H, 4 * H), full),
            pl.BlockSpec((2 * H, 4 * H), full),
            pl.BlockSpec((H, 4 * H), full),
            pl.BlockSpec((H, 4 * H), full),
            pl.BlockSpec((1, 4 * H), full),
            pl.BlockSpec((1, 4 * H), full),
        ],
        out_specs=pl.BlockSpec((BLK, H), lambda i: (i, 0)),
        out_shape=jax.ShapeDtypeStruct((N, H), jnp.float32),
        scratch_shapes=[pltpu.VMEM((BLK, L, H), jnp.float32),
                        pltpu.VMEM((BLK, L, H), jnp.float32)],
    )(
        emb, lengths.reshape(N, 1).astype(jnp.int32),
        lstm['l0f']['Wih'].T, lstm['l0b']['Wih'].T,
        lstm['l0f']['Whh'].T, lstm['l0b']['Whh'].T,
        (lstm['l0f']['bih'] + lstm['l0f']['bhh']).reshape(1, 4 * H),
        (lstm['l0b']['bih'] + lstm['l0b']['bhh']).reshape(1, 4 * H),
        lstm['l1f']['Wih'].T, lstm['l1b']['Wih'].T,
        lstm['l1f']['Whh'].T, lstm['l1b']['Whh'].T,
        (lstm['l1f']['bih'] + lstm['l1f']['bhh']).reshape(1, 4 * H),
        (lstm['l1b']['bih'] + lstm['l1b']['bhh']).reshape(1, 4 * H),
    )


EB = 8


def _edge_body(dst_hbm, src_hbm, eb_hbm, asrc_hbm, adst_hbm, xw_hbm, out_hbm,
               ebrow, src0, src1, dst0, dst1, ars0, ars1, ard0, ard1,
               msg0, msg1, slab, den, abuf,
               ssrc0, ssrc1, sdst0, sdst1, sgs0, sgs1, sgd0, sgd1, sm0, sm1):
    cid = lax.axis_index("c")
    sid = lax.axis_index("s")
    wid = sid * 2 + cid
    lane = lax.iota(jnp.int32, 16)
    zero16 = jnp.zeros((16,), jnp.float32)
    slots = ((src0, dst0, ars0, ard0, msg0, ssrc0, sdst0, sgs0, sgd0, sm0),
             (src1, dst1, ars1, ard1, msg1, ssrc1, sdst1, sgs1, sgd1, sm1))


    def edge_softmax_terms(j, s):
        s_lo = s[2][j, 0:16]
        s_hi = s[2][j, 16:32]
        d_lo = s[3][j, 0:16]
        d_hi = s[3][j, 16:32]
        e_lo = s_lo + d_lo
        e_hi = s_hi + d_hi
        e_lo = jnp.where(e_lo < 0.0, e_lo * 0.2, e_lo)
        e_hi = jnp.where(e_hi < 0.0, e_hi * 0.2, e_hi)
        return jnp.exp(e_lo), jnp.exp(e_hi)

    def range_body(rr, _):
        r = rr * NW + wid
        base = r * RNG
        pltpu.sync_copy(eb_hbm.at[r], ebrow)
        ebv = ebrow[...]
        e_start = ebv[0]
        e_end = ebv[1]
        k0 = (e_start // EB) * EB
        nb = jnp.maximum(e_end - k0 + EB - 1, 0) // EB

        def zrow(i, _):
            def zc(c, _):
                slab[i, pl.ds(c * 16, 16)] = zero16
                return 0
            lax.fori_loop(0, MW // 16, zc, 0)
            den[i, 0:16] = zero16
            den[i, 16:32] = zero16
            return 0
        lax.fori_loop(0, RNG, zrow, 0)

        def dloc_of(j, s):
            dsp = plsc.load_gather(s[1], [jnp.full((16,), j, jnp.int32)])
            return dsp & (RNG - 1)

        def run_pass(want_msg, process):
            def stage_idx(b, s):
                e0 = k0 + b * EB
                pltpu.async_copy(src_hbm.at[pl.ds(e0, EB)], s[0], s[5])
                pltpu.async_copy(dst_hbm.at[pl.ds(e0, EB)], s[1], s[6])

            def wait_idx(s):
                pltpu.make_async_copy(src_hbm.at[pl.ds(0, EB)], s[0], s[5]).wait()
                pltpu.make_async_copy(dst_hbm.at[pl.ds(0, EB)], s[1], s[6]).wait()

            def stage_gath(s):
                pltpu.async_copy(asrc_hbm.at[s[0]], s[2], s[7])
                pltpu.async_copy(adst_hbm.at[s[1]], s[3], s[8])
                if want_msg:
                    pltpu.async_copy(xw_hbm.at[s[0]], s[4], s[9])

            def wait_gath(s):
                pltpu.make_async_copy(asrc_hbm.at[s[0]], s[2], s[7]).wait()
                pltpu.make_async_copy(adst_hbm.at[s[1]], s[3], s[8]).wait()
                if want_msg:
                    pltpu.make_async_copy(xw_hbm.at[s[0]], s[4], s[9]).wait()

            @pl.when(nb > 0)
            def _():
                stage_idx(0, slots[0])
                wait_idx(slots[0])
                stage_gath(slots[0])

            @pl.when(nb > 1)
            def _():
                stage_idx(1, slots[1])

            def iter2(bb, _):
                for par in (0, 1):
                    b = bb * 2 + par
                    s = slots[par]
                    s2 = slots[1 - par]

                    @pl.when(b < nb)
                    def _():
                        wait_gath(s)

                        @pl.when(b + 1 < nb)
                        def _():
                            wait_idx(s2)
                            stage_gath(s2)

                        process(b, s)

                        @pl.when(b + 2 < nb)
                        def _():
                            stage_idx(b + 2, s)
                return 0
            lax.fori_loop(0, (nb + 1) // 2, iter2, 0)

        def proc_a(b, s):
            e0 = k0 + b * EB

            def pe(j, _):
                valid = (e0 + j >= e_start) & (e0 + j < e_end)
                ex_lo, ex_hi = edge_softmax_terms(j, s)
                vf = jnp.where(valid, 1.0, 0.0)
                dloc_v = dloc_of(j, s)
                plsc.addupdate_scatter(den, [dloc_v, lane], ex_lo * vf)
                plsc.addupdate_scatter(den, [dloc_v, lane + 16], ex_hi * vf)
                return 0
            lax.fori_loop(0, EB, pe, 0)

        run_pass(False, proc_a)

        def proc_b(b, s):
            e0 = k0 + b * EB

            def pe(j, _):
                valid = (e0 + j >= e_start) & (e0 + j < e_end)
                ex_lo, ex_hi = edge_softmax_terms(j, s)
                vf = jnp.where(valid, 1.0, 0.0)
                dloc_v = dloc_of(j, s)
                den_lo = plsc.load_gather(den, [dloc_v, lane])
                den_hi = plsc.load_gather(den, [dloc_v, lane + 16])
                abuf[0:16] = ex_lo / (den_lo + 1e-16) * vf
                abuf[16:32] = ex_hi / (den_hi + 1e-16) * vf
                for g in range(GH):
                    a_g = plsc.load_gather(abuf, [jnp.full((16,), g, jnp.int32)])
                    for c in range(H // 16):
                        off = g * H + c * 16
                        plsc.addupdate_scatter(
                            slab, [dloc_v, lane + off],
                            s[4][j, pl.ds(off, 16)] * a_g)
                return 0
            lax.fori_loop(0, EB, pe, 0)

        run_pass(True, proc_b)

        @pl.when(base < N)
        def _flush():
            pltpu.sync_copy(slab, out_hbm.at[pl.ds(base, RNG)])
        return 0

    lax.fori_loop(0, RPT, range_body, 0)


_edge_kernel = functools.partial(
    pl.kernel,
    out_type=jax.ShapeDtypeStruct((N, MW), jnp.float32),
    mesh=plsc.VectorSubcoreMesh(core_axis_name="c", subcore_axis_name="s"),
    compiler_params=pltpu.CompilerParams(needs_layout_passes=False),
    scratch_types=(
        [pltpu.VMEM((16,), jnp.int32)]
        + [pltpu.VMEM((EB,), jnp.int32)] * 4
        + [pltpu.VMEM((EB, 128), jnp.float32)] * 4
        + [pltpu.VMEM((EB, MW), jnp.float32)] * 2
        + [pltpu.VMEM((RNG, MW), jnp.float32)]
        + [pltpu.VMEM((RNG, 32), jnp.float32)]
        + [pltpu.VMEM((32,), jnp.float32)]
        + [pltpu.SemaphoreType.DMA] * 10
    ),
)(_edge_body)


def _gat3_sparse(node, edge_index, p1, p2, p3):
    src = edge_index[0]
    dst = edge_index[1]
    order = jnp.argsort(dst)
    dst_s = jnp.concatenate([dst[order], jnp.zeros((16,), dst.dtype)])
    src_s = jnp.concatenate([src[order], jnp.zeros((16,), src.dtype)])
    bounds = jnp.searchsorted(dst_s[:E_EDGES], jnp.arange(NRANGES + 1) * RNG)
    ebmat = jnp.zeros((NRANGES, 16), jnp.int32)
    ebmat = ebmat.at[:, 0].set(bounds[:NRANGES].astype(jnp.int32))
    ebmat = ebmat.at[:, 1].set(bounds[1:].astype(jnp.int32))

    xws = []
    asrcs = []
    adsts = []
    for p in (p1, p2, p3):
        xw = node @ p['W']
        xwh = xw.reshape(N, HEADS, H)
        asrcs.append((xwh * p['att_src'][None]).sum(-1))
        adsts.append((xwh * p['att_dst'][None]).sum(-1))
        xws.append(xw)
    xw_cat = jnp.concatenate(xws, axis=1)
    pad = jnp.zeros((N, 128 - GH), jnp.float32)
    asrc_cat = jnp.concatenate(asrcs + [pad], axis=1)
    adst_cat = jnp.concatenate(adsts + [pad], axis=1)

    out_cat = _edge_kernel(dst_s.astype(jnp.int32), src_s.astype(jnp.int32),
                           ebmat, asrc_cat, adst_cat, xw_cat)
    outs = []
    for i, p in enumerate((p1, p2, p3)):
        outs.append(jax.nn.relu(out_cat[:, i * HEADS * H:(i + 1) * HEADS * H]
                                + p['bias']))
    return outs


def _head_body(pooled_ref, w_ref, b_ref, labels_ref, logits_ref, loss_ref):
    logits = jnp.dot(pooled_ref[...], w_ref[...],
                     preferred_element_type=jnp.float32) + b_ref[...]
    m = jnp.max(logits, axis=-1, keepdims=True)
    lse = m + jnp.log(jnp.sum(jnp.exp(logits - m), axis=-1, keepdims=True))
    logp = logits - lse
    cols = lax.broadcasted_iota(jnp.int32, logp.shape, 1)
    pick = jnp.sum(jnp.where(cols == labels_ref[...], logp, 0.0), axis=-1)
    loss_ref[...] = jnp.reshape(-jnp.mean(pick), (1, 1))
    logits_ref[...] = logits


def _head(pooled, w, b, labels):
    logits, loss = pl.pallas_call(
        _head_body,
        out_shape=(
            jax.ShapeDtypeStruct((G, C), jnp.float32),
            jax.ShapeDtypeStruct((1, 1), jnp.float32),
        ),
    )(pooled, w, b.reshape(1, C), labels.reshape(G, 1).astype(jnp.int32))
    return loss.reshape(()), logits


def kernel(params, x, edge_index, batch, labels):
    input_ids = x[:, 0, :]
    attn = x[:, 1, :]
    lengths = attn.sum(axis=-1)
    emb = jnp.take(params['emb'], input_ids, axis=0)
    node = _bilstm_node(emb, lengths, params['lstm'])
    sems = _gat3_sparse(node, edge_index, params['gat1'], params['gat2'], params['gat3'])
    z = jnp.stack(sems, axis=1)
    w = jnp.tanh(z @ params['sem']['W1'] + params['sem']['b1']) @ params['sem']['W2']
    beta = jax.nn.softmax(w.mean(axis=0), axis=0)
    sem_emb = (beta[None] * z).sum(axis=1)
    ng = labels.shape[0]
    sums = jax.ops.segment_sum(sem_emb, batch, num_segments=ng)
    cnt = jax.ops.segment_sum(jnp.ones((sem_emb.shape[0],), sem_emb.dtype), batch, num_segments=ng)
    pooled = sums / jnp.clip(cnt, 1.0)[:, None]
    loss, logits = _head(pooled, params['cls']['W'], params['cls']['b'], labels)
    return loss, logits
